# Initial kernel scaffold; baseline (speedup 1.0000x reference)
import jax
import jax.numpy as jnp
from jax.experimental import pallas as pl


def kernel(x, edge_index_g1, mask_g1, edge_index_g2, mask_g2, edge_index_g3_u, mask_g3_u, edge_index_g3_q, mask_g3_q, edge_index_g4_u, mask_g4_u, edge_index_g4_q, mask_g4_q, Wg1, Wg2, Wg3, Wg4, p1_c0_wpl, p1_c0_wpr, p1_c0_bpr, p1_c0_wnl, p1_c0_wnr, p1_c0_bnr, p1_c1_wpl, p1_c1_wpr, p1_c1_bpr, p1_c1_wnl, p1_c1_wnr, p1_c1_bnr, p2_c0_wpl, p2_c0_wpr, p2_c0_bpr, p2_c0_wnl, p2_c0_wnr, p2_c0_bnr, p2_c1_wpl, p2_c1_wpr, p2_c1_bpr, p2_c1_wnl, p2_c1_wnr, p2_c1_bnr):
    raise NotImplementedError("write your pallas kernel here")


# speedup vs baseline: 9.6670x; 9.6670x over previous
import functools

import jax
import jax.numpy as jnp
from jax import lax
from jax.experimental import pallas as pl
from jax.experimental.pallas import tpu as pltpu
from jax.experimental.pallas import tpu_sc as plsc

N = 10000
H = 64
E = 320000
NV = 6
CH = 125
NSLOT = 4
NSTG = 4
NSTG_CH = 20
NCORE = 2
NSUB = 16
NCHUNK = 80
CPV = E // CH
ACCR = 20096
ROWS_PER_SUB = ACCR // NSUB
CNTP = 20480
CSLICE = CNTP // NSUB
PGL = (0, 1, 2, 2, 3, 3)


def _tc_proj(x, w8):
    def body(x_ref, w_ref, o_ref):
        o_ref[0] = lax.dot(x_ref[...], w_ref[0],
                           preferred_element_type=jnp.float32)
    return pl.pallas_call(
        body,
        grid=(8,),
        in_specs=[
            pl.BlockSpec((N, 128), lambda g: (0, 0)),
            pl.BlockSpec((1, 128, H), lambda g: (g, 0, 0)),
        ],
        out_specs=pl.BlockSpec((1, N, H), lambda g: (g, 0, 0)),
        out_shape=jax.ShapeDtypeStruct((8, N, H), jnp.float32),
    )(x, w8)


def _tc_indices(ei6, m6):
    B = 32000

    def body(ei_ref, m_ref, g1_ref, g2_ref, dk_ref):
        for v in range(NV):
            src = ei_ref[v, 0]
            dst = ei_ref[v, 1]
            m = m_ref[v]
            dk_ref[v] = jnp.where(m == 1, dst, dst + N)
            for h in range(2):
                g1_ref[h, v] = src + (2 * PGL[v] + h) * N
                g2_ref[h, v] = src + (2 * v + h) * N

    nb = E // B
    return pl.pallas_call(
        body,
        grid=(nb,),
        in_specs=[
            pl.BlockSpec((NV, 2, B), lambda i: (0, 0, i)),
            pl.BlockSpec((NV, B), lambda i: (0, i)),
        ],
        out_specs=[
            pl.BlockSpec((2, NV, B), lambda i: (0, 0, i)),
            pl.BlockSpec((2, NV, B), lambda i: (0, 0, i)),
            pl.BlockSpec((NV, B), lambda i: (0, i)),
        ],
        out_shape=[
            jax.ShapeDtypeStruct((2, NV, E), jnp.int32),
            jax.ShapeDtypeStruct((2, NV, E), jnp.int32),
            jax.ShapeDtypeStruct((NV, E), jnp.int32),
        ],
    )(ei6, m6)


def _tc_layer1(part, cnt, xlr, wpl, wnl, wpr, wnr, bp, bn):
    R = 1000

    def body(pp, pn, cp_r, cn_r, xl_r, xr_r, wpl_r, wnl_r, wpr_r, wnr_r,
             bp_r, bn_r, z_ref):
        dot = lambda a, b: lax.dot(a, b, preferred_element_type=jnp.float32)
        rp = 1.0 / jnp.maximum(cp_r[0, 0, 0] + cp_r[1, 0, 0], 1.0)
        rn = 1.0 / jnp.maximum(cn_r[0, 0, 0] + cn_r[1, 0, 0], 1.0)
        mpL = (pp[0, 0, 0] + pp[1, 0, 0]) * rp
        mpR = (pp[0, 0, 1] + pp[1, 0, 1]) * rp
        mnL = (pn[0, 0, 0] + pn[1, 0, 0]) * rn
        mnR = (pn[0, 0, 1] + pn[1, 0, 1]) * rn
        XL, XR = xl_r[0], xr_r[0]
        wl_p, wl_n, wr_p, wr_n = wpl_r[0], wnl_r[0], wpr_r[0], wnr_r[0]
        zL = (dot(mpL, wl_p[:H]) + dot(mpR, wl_p[H:])
              + dot(XL, wr_p[:H]) + dot(XR, wr_p[H:]) + bp_r[0, 0])
        zR = (dot(mnL, wl_n[:H]) + dot(mnR, wl_n[H:])
              + dot(XL, wr_n[:H]) + dot(XR, wr_n[H:]) + bn_r[0, 0])
        z_ref[0, 0] = jnp.maximum(zL, 0.0)
        z_ref[0, 1] = jnp.maximum(zR, 0.0)

    def pg(v):
        return jnp.where(v < 2, v, (v + 2) // 2)

    return pl.pallas_call(
        body,
        grid=(NV, N // R),
        in_specs=[
            pl.BlockSpec((NCORE, 1, 2, R, H), lambda v, r: (0, v, 0, r, 0)),
            pl.BlockSpec((NCORE, 1, 2, R, H), lambda v, r: (0, v, 0, 10 + r, 0)),
            pl.BlockSpec((NCORE, 1, 1, R, 1), lambda v, r: (0, v, 0, r, 0)),
            pl.BlockSpec((NCORE, 1, 1, R, 1), lambda v, r: (0, v, 1, r, 0)),
            pl.BlockSpec((1, R, H), lambda v, r: (2 * pg(v), r, 0)),
            pl.BlockSpec((1, R, H), lambda v, r: (2 * pg(v) + 1, r, 0)),
            pl.BlockSpec((1, 128, H), lambda v, r: (v, 0, 0)),
            pl.BlockSpec((1, 128, H), lambda v, r: (v, 0, 0)),
            pl.BlockSpec((1, 128, H), lambda v, r: (v, 0, 0)),
            pl.BlockSpec((1, 128, H), lambda v, r: (v, 0, 0)),
            pl.BlockSpec((1, 1, H), lambda v, r: (v, 0, 0)),
            pl.BlockSpec((1, 1, H), lambda v, r: (v, 0, 0)),
        ],
        out_specs=pl.BlockSpec((1, 2, R, H), lambda v, r: (v, 0, r, 0)),
        out_shape=jax.ShapeDtypeStruct((NV, 2, N, H), jnp.float32),
    )(part, part, cnt, cnt, xlr, xlr, wpl, wnl, wpr, wnr, bp, bn)


def _tc_layer2(part, cnt, zst, wpl, wnl, wpr, wnr, bp, bn):
    R = 1000

    def body(pp, pn, cp_r, cn_r, zl_r, zr_r, wpl_r, wnl_r, wpr_r, wnr_r,
             bp_r, bn_r, o_ref):
        dot = lambda a, b: lax.dot(a, b, preferred_element_type=jnp.float32)
        rp = 1.0 / jnp.maximum(cp_r[0, 0, 0] + cp_r[1, 0, 0], 1.0)
        rn = 1.0 / jnp.maximum(cn_r[0, 0, 0] + cn_r[1, 0, 0], 1.0)
        MpL = (pp[0, 0, 0] + pp[1, 0, 0]) * rp
        MpR = (pp[0, 0, 1] + pp[1, 0, 1]) * rp
        MnL = (pn[0, 0, 0] + pn[1, 0, 0]) * rn
        MnR = (pn[0, 0, 1] + pn[1, 0, 1]) * rn
        zL, zR = zl_r[0, 0], zr_r[0, 0]
        wl_p, wl_n, wr_p, wr_n = wpl_r[0], wnl_r[0], wpr_r[0], wnr_r[0]
        op = dot(MpL, wl_p[:H]) + dot(MnR, wl_p[H:]) + dot(zL, wr_p) + bp_r[0, 0]
        on = dot(MpR, wl_n[:H]) + dot(MnL, wl_n[H:]) + dot(zR, wr_n) + bn_r[0, 0]
        o_ref[0] = jnp.maximum(
            jnp.concatenate([op, on], axis=1), 0.0)

    return pl.pallas_call(
        body,
        grid=(NV, N // R),
        in_specs=[
            pl.BlockSpec((NCORE, 1, 2, R, H), lambda v, r: (0, v, 0, r, 0)),
            pl.BlockSpec((NCORE, 1, 2, R, H), lambda v, r: (0, v, 0, 10 + r, 0)),
            pl.BlockSpec((NCORE, 1, 1, R, 1), lambda v, r: (0, v, 0, r, 0)),
            pl.BlockSpec((NCORE, 1, 1, R, 1), lambda v, r: (0, v, 1, r, 0)),
            pl.BlockSpec((1, 1, R, H), lambda v, r: (v, 0, r, 0)),
            pl.BlockSpec((1, 1, R, H), lambda v, r: (v, 1, r, 0)),
            pl.BlockSpec((1, 128, H), lambda v, r: (v, 0, 0)),
            pl.BlockSpec((1, 128, H), lambda v, r: (v, 0, 0)),
            pl.BlockSpec((1, H, H), lambda v, r: (v, 0, 0)),
            pl.BlockSpec((1, H, H), lambda v, r: (v, 0, 0)),
            pl.BlockSpec((1, 1, H), lambda v, r: (v, 0, 0)),
            pl.BlockSpec((1, 1, H), lambda v, r: (v, 0, 0)),
        ],
        out_specs=pl.BlockSpec((1, R, 2 * H), lambda v, r: (v, r, 0)),
        out_shape=jax.ShapeDtypeStruct((NV, N, 2 * H), jnp.float32),
    )(part, part, cnt, cnt, zst, zst, wpl, wnl, wpr, wnr, bp, bn)


def _make_agg(tab_rows, do_count):
    mesh = plsc.VectorSubcoreMesh(core_axis_name="c", subcore_axis_name="s")
    out_type = [jax.ShapeDtypeStruct((NCORE, NV, 2, ACCR, H), jnp.float32)]
    scratch = [pltpu.VMEM((NSTG_CH, CH), jnp.int32) for _ in range(4)]
    scratch += [pltpu.VMEM((CH, H), jnp.float32) for _ in range(NSLOT)]
    scratch += [pltpu.VMEM_SHARED((ACCR, H), jnp.float32)]
    scratch += [pltpu.SemaphoreType.DMA for _ in range(2 * NSLOT)]
    if do_count:
        out_type.append(jax.ShapeDtypeStruct(
            (NCORE, NV, NSUB, 1, CSLICE), jnp.float32))
        scratch += [
            pltpu.VMEM((128,), jnp.float32),
            pltpu.VMEM_SHARED((CNTP,), jnp.float32),
        ]
        scratch += [pltpu.SemaphoreType.DMA for _ in range(NSLOT)]

    def body(tab, gidx, dstk, *rest):
        if do_count:
            (z2d, z1d, part, cnt, ga, gb, da, db, r0, r1, r2, r3, acc,
             g0, g1, g2, g3, s0, s1, s2, s3,
             ones, cntacc, c0, c1, c2, c3) = rest
            scs = [c0, c1, c2, c3]
        else:
            (z2d, part, ga, gb, da, db, r0, r1, r2, r3, acc,
             g0, g1, g2, g3, s0, s1, s2, s3) = rest
        rows = [r0, r1, r2, r3]
        sgs = [g0, g1, g2, g3]
        sss = [s0, s1, s2, s3]
        cid = lax.axis_index("c")
        sid = lax.axis_index("s")

        if do_count:
            one16 = jnp.ones((16,), jnp.float32)
            for q in range(8):
                ones[pl.ds(q * 16, 16)] = one16

        def do_pass(vh, carry):
            v = vh // 2
            h = vh - v * 2

            pltpu.sync_copy(z2d, acc.at[pl.ds(sid * ROWS_PER_SUB,
                                              ROWS_PER_SUB)])
            if do_count:
                @pl.when(h == 0)
                def _():
                    pltpu.sync_copy(z1d,
                                    cntacc.at[pl.ds(sid * CSLICE, CSLICE)])
            plsc.subcore_barrier()

            gchunk = (h * NV + v) * CPV + cid * (CPV // 2) + sid * NCHUNK
            dchunk = v * CPV + cid * (CPV // 2) + sid * NCHUNK

            def issue_scatter(m, s, qd):
                pltpu.async_copy(rows[s], acc.at[qd.at[m]], sss[s], add=True)
                if do_count:
                    @pl.when(h == 0)
                    def _():
                        pltpu.async_copy(ones.at[pl.ds(0, CH)],
                                         cntacc.at[qd.at[m]],
                                         scs[s], add=True)

            def wait_scatter(s, qd):
                pltpu.make_async_copy(rows[s], acc.at[qd.at[0]],
                                      sss[s]).wait()
                if do_count:
                    @pl.when(h == 0)
                    def _():
                        pltpu.make_async_copy(ones.at[pl.ds(0, CH)],
                                              cntacc.at[qd.at[0]],
                                              scs[s]).wait()

            def wait_gather(s, qg):
                pltpu.make_async_copy(tab.at[qg.at[0]], rows[s],
                                      sgs[s]).wait()

            for s4 in range(NSTG):
                qg = (ga, gb)[s4 % 2]
                qd = (da, db)[s4 % 2]
                pltpu.sync_copy(
                    gidx.at[pl.ds(gchunk + s4 * NSTG_CH, NSTG_CH)], qg)
                pltpu.sync_copy(
                    dstk.at[pl.ds(dchunk + s4 * NSTG_CH, NSTG_CH)], qd)

                def quad(qq, c, s4=s4, qg=qg, qd=qd):
                    for s in range(NSLOT):
                        m = qq * NSLOT + s
                        jg = s4 * NSTG_CH + m

                        @pl.when(jg >= NSLOT)
                        def _(s=s, qd=qd):
                            wait_scatter(s, qd)
                        pltpu.async_copy(tab.at[qg.at[m]], rows[s], sgs[s])

                        @pl.when(m >= 1)
                        def _(m=m, s=s, qg=qg, qd=qd):
                            wait_gather((s - 1) % NSLOT, qg)
                            issue_scatter(m - 1, (s - 1) % NSLOT, qd)
                    return c
                lax.fori_loop(0, NSTG_CH // NSLOT, quad, 0)
                wait_gather(NSLOT - 1, qg)
                issue_scatter(NSTG_CH - 1, NSLOT - 1, qd)

            for s in range(NSLOT):
                wait_scatter(s, db)
            plsc.subcore_barrier()

            rbase = sid * ROWS_PER_SUB
            pltpu.sync_copy(
                acc.at[pl.ds(rbase, ROWS_PER_SUB)],
                part.at[cid, v, h, pl.ds(rbase, ROWS_PER_SUB)])
            if do_count:
                @pl.when(h == 0)
                def _():
                    pltpu.sync_copy(
                        cntacc.at[pl.ds(sid * CSLICE, CSLICE)],
                        cnt.at[cid, v, sid, 0])
            return carry

        lax.fori_loop(0, NV * 2, do_pass, 0)

    return pl.kernel(
        body, out_type=out_type, mesh=mesh, scratch_types=scratch,
        compiler_params=pltpu.CompilerParams(use_tc_tiling_on_sc=False))


def kernel(x, edge_index_g1, mask_g1, edge_index_g2, mask_g2,
           edge_index_g3_u, mask_g3_u, edge_index_g3_q, mask_g3_q,
           edge_index_g4_u, mask_g4_u, edge_index_g4_q, mask_g4_q,
           Wg1, Wg2, Wg3, Wg4,
           p1_c0_wpl, p1_c0_wpr, p1_c0_bpr, p1_c0_wnl, p1_c0_wnr, p1_c0_bnr,
           p1_c1_wpl, p1_c1_wpr, p1_c1_bpr, p1_c1_wnl, p1_c1_wnr, p1_c1_bnr,
           p2_c0_wpl, p2_c0_wpr, p2_c0_bpr, p2_c0_wnl, p2_c0_wnr, p2_c0_bnr,
           p2_c1_wpl, p2_c1_wpr, p2_c1_bpr, p2_c1_wnl, p2_c1_wnr, p2_c1_bnr):
    f32 = jnp.float32
    eis = [edge_index_g1, edge_index_g2, edge_index_g3_u,
           edge_index_g3_q, edge_index_g4_u, edge_index_g4_q]
    masks = [mask_g1, mask_g2, mask_g3_u, mask_g3_q, mask_g4_u, mask_g4_q]
    ei6 = jnp.stack(eis)
    m6 = jnp.stack(masks).astype(jnp.int32)

    w8 = jnp.stack([W[:, h * H:(h + 1) * H]
                    for W in (Wg1, Wg2, Wg3, Wg4) for h in (0, 1)])

    def stack6(a1, a2):
        return jnp.stack([a1, a1, a2, a2, a2, a2])

    def stack6b(a1, a2):
        return jnp.stack([a1, a1, a2, a2, a2, a2])[:, None, :]

    wpl1 = stack6(p1_c0_wpl, p2_c0_wpl)
    wnl1 = stack6(p1_c0_wnl, p2_c0_wnl)
    wpr1 = stack6(p1_c0_wpr, p2_c0_wpr)
    wnr1 = stack6(p1_c0_wnr, p2_c0_wnr)
    bp1 = stack6b(p1_c0_bpr, p2_c0_bpr)
    bn1 = stack6b(p1_c0_bnr, p2_c0_bnr)
    wpl2 = stack6(p1_c1_wpl, p2_c1_wpl)
    wnl2 = stack6(p1_c1_wnl, p2_c1_wnl)
    wpr2 = stack6(p1_c1_wpr, p2_c1_wpr)
    wnr2 = stack6(p1_c1_wnr, p2_c1_wnr)
    bp2 = stack6b(p1_c1_bpr, p2_c1_bpr)
    bn2 = stack6b(p1_c1_bnr, p2_c1_bnr)

    xlr = _tc_proj(x.astype(f32), w8.astype(f32))
    gidx1, gidx2, dstk = _tc_indices(ei6, m6)
    gidx1 = gidx1.reshape(-1, CH)
    gidx2 = gidx2.reshape(-1, CH)
    dstk = dstk.reshape(-1, CH)

    z2d = jnp.zeros((ROWS_PER_SUB, H), f32)
    z1d = jnp.zeros((CSLICE,), f32)
    agg1 = _make_agg(8 * N, True)
    part1, cnt = agg1(xlr.reshape(8 * N, H), gidx1, dstk, z2d, z1d)
    cnt5 = cnt.reshape(NCORE, NV, CNTP)[:, :, :2 * N]
    cnt5 = cnt5.reshape(NCORE, NV, 2, N, 1)
    zst = _tc_layer1(part1, cnt5, xlr, wpl1, wnl1, wpr1, wnr1, bp1, bn1)

    agg2 = _make_agg(2 * NV * N, False)
    part2, = agg2(zst.reshape(2 * NV * N, H), gidx2, dstk, z2d)
    outs = _tc_layer2(part2, cnt5, zst, wpl2, wnl2, wpr2, wnr2, bp2, bn2)

    e1, e2 = outs[0], outs[1]
    e3 = jnp.concatenate([outs[2][:6000], outs[3][6000:]], axis=0)
    e4 = jnp.concatenate([outs[4][:6000], outs[5][6000:]], axis=0)
    return (e1, e2, e3, e4)


# speedup vs baseline: 11.5030x; 1.1899x over previous
import jax
import jax.numpy as jnp
from jax import lax
from jax.experimental import pallas as pl
from jax.experimental.pallas import tpu as pltpu
from jax.experimental.pallas import tpu_sc as plsc

N = 10000
H = 64
E = 320000
NV = 6
NB = 3
CH = 125
NSLOT = 4
NSTG = 4
NSTG_CH = 20
NCORE = 2
NSUB = 16
NCHUNK = 80
CPV = E // CH
ACCR = 20096
ROWS_PER_SUB = ACCR // NSUB
CNTP = 20480
CSLICE = CNTP // NSUB
PGL = (0, 1, 2, 2, 3, 3)


def _tc_proj(x, w8):
    def body(x_ref, w_ref, o_ref):
        o_ref[0] = lax.dot(x_ref[...], w_ref[0],
                           preferred_element_type=jnp.float32)
    return pl.pallas_call(
        body,
        grid=(8,),
        in_specs=[
            pl.BlockSpec((N, 128), lambda g: (0, 0)),
            pl.BlockSpec((1, 128, H), lambda g: (g, 0, 0)),
        ],
        out_specs=pl.BlockSpec((1, N, H), lambda g: (g, 0, 0)),
        out_shape=jax.ShapeDtypeStruct((8, N, H), jnp.float32),
    )(x, w8)


def _tc_indices(eis, masks):
    B = 32000

    def body(*refs):
        ei_refs = refs[0:NV]
        m_refs = refs[NV:2 * NV]
        g1a, g1b, g2a, g2b, dka, dkb = refs[2 * NV:]
        for v in range(NV):
            blk, vl = divmod(v, NB)
            g1, g2, dk = ((g1a, g2a, dka), (g1b, g2b, dkb))[blk]
            src = ei_refs[v][0]
            dst = ei_refs[v][1]
            m = m_refs[v][0]
            dk[vl] = jnp.where(m == 1, dst, dst + N)
            for h in range(2):
                g1[h, vl] = src + (2 * PGL[v] + h) * N
                g2[h, vl] = src + (2 * vl + h) * N

    nb = E // B
    io = jax.ShapeDtypeStruct((2, NB, E), jnp.int32)
    do = jax.ShapeDtypeStruct((NB, E), jnp.int32)
    return pl.pallas_call(
        body,
        grid=(nb,),
        in_specs=[pl.BlockSpec((2, B), lambda i: (0, i))] * NV
        + [pl.BlockSpec((1, B), lambda i: (0, i))] * NV,
        out_specs=[pl.BlockSpec((2, NB, B), lambda i: (0, 0, i))] * 4
        + [pl.BlockSpec((NB, B), lambda i: (0, i))] * 2,
        out_shape=[io, io, io, io, do, do],
    )(*eis, *masks)


def _mk_pg(pgl):
    def f(v):
        r = pgl[0]
        for i in range(1, len(pgl)):
            r = r + (pgl[i] - pgl[i - 1]) * (v >= i)
        return r
    return f


def _tc_layer1(part, cnt, xlr, wpl, wnl, wpr, wnr, bp, bn, pgl):
    R = 1000
    pg = _mk_pg(pgl)

    def body(pp, pn, cp_r, cn_r, xl_r, xr_r, wpl_r, wnl_r, wpr_r, wnr_r,
             bp_r, bn_r, z_ref):
        dot = lambda a, b: lax.dot(a, b, preferred_element_type=jnp.float32)
        rp = 1.0 / jnp.maximum(cp_r[0, 0, 0] + cp_r[1, 0, 0], 1.0)
        rn = 1.0 / jnp.maximum(cn_r[0, 0, 0] + cn_r[1, 0, 0], 1.0)
        mpL = (pp[0, 0, 0] + pp[1, 0, 0]) * rp
        mpR = (pp[0, 0, 1] + pp[1, 0, 1]) * rp
        mnL = (pn[0, 0, 0] + pn[1, 0, 0]) * rn
        mnR = (pn[0, 0, 1] + pn[1, 0, 1]) * rn
        XL, XR = xl_r[0], xr_r[0]
        wl_p, wl_n, wr_p, wr_n = wpl_r[0], wnl_r[0], wpr_r[0], wnr_r[0]
        zL = (dot(mpL, wl_p[:H]) + dot(mpR, wl_p[H:])
              + dot(XL, wr_p[:H]) + dot(XR, wr_p[H:]) + bp_r[0, 0])
        zR = (dot(mnL, wl_n[:H]) + dot(mnR, wl_n[H:])
              + dot(XL, wr_n[:H]) + dot(XR, wr_n[H:]) + bn_r[0, 0])
        z_ref[0, 0] = jnp.maximum(zL, 0.0)
        z_ref[0, 1] = jnp.maximum(zR, 0.0)

    return pl.pallas_call(
        body,
        grid=(NB, N // R),
        in_specs=[
            pl.BlockSpec((NCORE, 1, 2, R, H), lambda v, r: (0, v, 0, r, 0)),
            pl.BlockSpec((NCORE, 1, 2, R, H), lambda v, r: (0, v, 0, 10 + r, 0)),
            pl.BlockSpec((NCORE, 1, 1, R, 1), lambda v, r: (0, v, 0, r, 0)),
            pl.BlockSpec((NCORE, 1, 1, R, 1), lambda v, r: (0, v, 1, r, 0)),
            pl.BlockSpec((1, R, H), lambda v, r: (2 * pg(v), r, 0)),
            pl.BlockSpec((1, R, H), lambda v, r: (2 * pg(v) + 1, r, 0)),
            pl.BlockSpec((1, 128, H), lambda v, r: (v, 0, 0)),
            pl.BlockSpec((1, 128, H), lambda v, r: (v, 0, 0)),
            pl.BlockSpec((1, 128, H), lambda v, r: (v, 0, 0)),
            pl.BlockSpec((1, 128, H), lambda v, r: (v, 0, 0)),
            pl.BlockSpec((1, 1, H), lambda v, r: (v, 0, 0)),
            pl.BlockSpec((1, 1, H), lambda v, r: (v, 0, 0)),
        ],
        out_specs=pl.BlockSpec((1, 2, R, H), lambda v, r: (v, 0, r, 0)),
        out_shape=jax.ShapeDtypeStruct((NB, 2, N, H), jnp.float32),
    )(part, part, cnt, cnt, xlr, xlr, wpl, wnl, wpr, wnr, bp, bn)


def _tc_layer2(part, cnt, zst, wpl, wnl, wpr, wnr, bp, bn):
    R = 1000

    def body(pp, pn, cp_r, cn_r, zl_r, zr_r, wpl_r, wnl_r, wpr_r, wnr_r,
             bp_r, bn_r, o_ref):
        dot = lambda a, b: lax.dot(a, b, preferred_element_type=jnp.float32)
        rp = 1.0 / jnp.maximum(cp_r[0, 0, 0] + cp_r[1, 0, 0], 1.0)
        rn = 1.0 / jnp.maximum(cn_r[0, 0, 0] + cn_r[1, 0, 0], 1.0)
        MpL = (pp[0, 0, 0] + pp[1, 0, 0]) * rp
        MpR = (pp[0, 0, 1] + pp[1, 0, 1]) * rp
        MnL = (pn[0, 0, 0] + pn[1, 0, 0]) * rn
        MnR = (pn[0, 0, 1] + pn[1, 0, 1]) * rn
        zL, zR = zl_r[0, 0], zr_r[0, 0]
        wl_p, wl_n, wr_p, wr_n = wpl_r[0], wnl_r[0], wpr_r[0], wnr_r[0]
        op = dot(MpL, wl_p[:H]) + dot(MnR, wl_p[H:]) + dot(zL, wr_p) + bp_r[0, 0]
        on = dot(MpR, wl_n[:H]) + dot(MnL, wl_n[H:]) + dot(zR, wr_n) + bn_r[0, 0]
        o_ref[0] = jnp.maximum(jnp.concatenate([op, on], axis=1), 0.0)

    return pl.pallas_call(
        body,
        grid=(NB, N // R),
        in_specs=[
            pl.BlockSpec((NCORE, 1, 2, R, H), lambda v, r: (0, v, 0, r, 0)),
            pl.BlockSpec((NCORE, 1, 2, R, H), lambda v, r: (0, v, 0, 10 + r, 0)),
            pl.BlockSpec((NCORE, 1, 1, R, 1), lambda v, r: (0, v, 0, r, 0)),
            pl.BlockSpec((NCORE, 1, 1, R, 1), lambda v, r: (0, v, 1, r, 0)),
            pl.BlockSpec((1, 1, R, H), lambda v, r: (v, 0, r, 0)),
            pl.BlockSpec((1, 1, R, H), lambda v, r: (v, 1, r, 0)),
            pl.BlockSpec((1, 128, H), lambda v, r: (v, 0, 0)),
            pl.BlockSpec((1, 128, H), lambda v, r: (v, 0, 0)),
            pl.BlockSpec((1, H, H), lambda v, r: (v, 0, 0)),
            pl.BlockSpec((1, H, H), lambda v, r: (v, 0, 0)),
            pl.BlockSpec((1, 1, H), lambda v, r: (v, 0, 0)),
            pl.BlockSpec((1, 1, H), lambda v, r: (v, 0, 0)),
        ],
        out_specs=pl.BlockSpec((1, R, 2 * H), lambda v, r: (v, r, 0)),
        out_shape=jax.ShapeDtypeStruct((NB, N, 2 * H), jnp.float32),
    )(part, part, cnt, cnt, zst, zst, wpl, wnl, wpr, wnr, bp, bn)


def _make_agg(do_count):
    mesh = plsc.VectorSubcoreMesh(core_axis_name="c", subcore_axis_name="s")
    out_type = [jax.ShapeDtypeStruct((NCORE, NB, 2, ACCR, H), jnp.float32)]
    scratch = [pltpu.VMEM((NSTG_CH, CH), jnp.int32) for _ in range(4)]
    scratch += [pltpu.VMEM((CH, H), jnp.float32) for _ in range(NSLOT)]
    scratch += [pltpu.VMEM_SHARED((ACCR, H), jnp.float32)]
    scratch += [pltpu.SemaphoreType.DMA for _ in range(2 * NSLOT)]
    if do_count:
        out_type.append(jax.ShapeDtypeStruct(
            (NCORE, NB, NSUB, 1, CSLICE), jnp.float32))
        scratch += [
            pltpu.VMEM((128,), jnp.float32),
            pltpu.VMEM_SHARED((CNTP,), jnp.float32),
        ]
        scratch += [pltpu.SemaphoreType.DMA for _ in range(NSLOT)]

    def body(tab, gidx, dstk, *rest):
        if do_count:
            (z2d, z1d, part, cnt, ga, gb, da, db, r0, r1, r2, r3, acc,
             g0, g1, g2, g3, s0, s1, s2, s3,
             ones, cntacc, c0, c1, c2, c3) = rest
            scs = [c0, c1, c2, c3]
        else:
            (z2d, part, ga, gb, da, db, r0, r1, r2, r3, acc,
             g0, g1, g2, g3, s0, s1, s2, s3) = rest
        rows = [r0, r1, r2, r3]
        sgs = [g0, g1, g2, g3]
        sss = [s0, s1, s2, s3]
        cid = lax.axis_index("c")
        sid = lax.axis_index("s")

        if do_count:
            one16 = jnp.ones((16,), jnp.float32)
            for q in range(8):
                ones[pl.ds(q * 16, 16)] = one16

        def do_pass(vh, carry):
            v = vh // 2
            h = vh - v * 2

            pltpu.sync_copy(z2d, acc.at[pl.ds(sid * ROWS_PER_SUB,
                                              ROWS_PER_SUB)])
            if do_count:
                @pl.when(h == 0)
                def _():
                    pltpu.sync_copy(z1d,
                                    cntacc.at[pl.ds(sid * CSLICE, CSLICE)])
            plsc.subcore_barrier()

            gchunk = (h * NB + v) * CPV + cid * (CPV // 2) + sid * NCHUNK
            dchunk = v * CPV + cid * (CPV // 2) + sid * NCHUNK

            def issue_scatter(m, s, qd):
                pltpu.async_copy(rows[s], acc.at[qd.at[m]], sss[s], add=True)
                if do_count:
                    @pl.when(h == 0)
                    def _():
                        pltpu.async_copy(ones.at[pl.ds(0, CH)],
                                         cntacc.at[qd.at[m]],
                                         scs[s], add=True)

            def wait_scatter(s, qd):
                pltpu.make_async_copy(rows[s], acc.at[qd.at[0]],
                                      sss[s]).wait()
                if do_count:
                    @pl.when(h == 0)
                    def _():
                        pltpu.make_async_copy(ones.at[pl.ds(0, CH)],
                                              cntacc.at[qd.at[0]],
                                              scs[s]).wait()

            def wait_gather(s, qg):
                pltpu.make_async_copy(tab.at[qg.at[0]], rows[s],
                                      sgs[s]).wait()

            for s4 in range(NSTG):
                qg = (ga, gb)[s4 % 2]
                qd = (da, db)[s4 % 2]
                pltpu.sync_copy(
                    gidx.at[pl.ds(gchunk + s4 * NSTG_CH, NSTG_CH)], qg)
                pltpu.sync_copy(
                    dstk.at[pl.ds(dchunk + s4 * NSTG_CH, NSTG_CH)], qd)

                def quad(qq, c, s4=s4, qg=qg, qd=qd):
                    for s in range(NSLOT):
                        m = qq * NSLOT + s
                        jg = s4 * NSTG_CH + m

                        @pl.when(jg >= NSLOT)
                        def _(s=s, qd=qd):
                            wait_scatter(s, qd)
                        pltpu.async_copy(tab.at[qg.at[m]], rows[s], sgs[s])

                        @pl.when(m >= 1)
                        def _(m=m, s=s, qg=qg, qd=qd):
                            wait_gather((s - 1) % NSLOT, qg)
                            issue_scatter(m - 1, (s - 1) % NSLOT, qd)
                    return c
                lax.fori_loop(0, NSTG_CH // NSLOT, quad, 0)
                wait_gather(NSLOT - 1, qg)
                issue_scatter(NSTG_CH - 1, NSLOT - 1, qd)

            for s in range(NSLOT):
                wait_scatter(s, db)
            plsc.subcore_barrier()

            rbase = sid * ROWS_PER_SUB
            pltpu.sync_copy(
                acc.at[pl.ds(rbase, ROWS_PER_SUB)],
                part.at[cid, v, h, pl.ds(rbase, ROWS_PER_SUB)])
            if do_count:
                @pl.when(h == 0)
                def _():
                    pltpu.sync_copy(
                        cntacc.at[pl.ds(sid * CSLICE, CSLICE)],
                        cnt.at[cid, v, sid, 0])
            return carry

        lax.fori_loop(0, NB * 2, do_pass, 0)

    return pl.kernel(
        body, out_type=out_type, mesh=mesh, scratch_types=scratch,
        compiler_params=pltpu.CompilerParams(use_tc_tiling_on_sc=False))


def kernel(x, edge_index_g1, mask_g1, edge_index_g2, mask_g2,
           edge_index_g3_u, mask_g3_u, edge_index_g3_q, mask_g3_q,
           edge_index_g4_u, mask_g4_u, edge_index_g4_q, mask_g4_q,
           Wg1, Wg2, Wg3, Wg4,
           p1_c0_wpl, p1_c0_wpr, p1_c0_bpr, p1_c0_wnl, p1_c0_wnr, p1_c0_bnr,
           p1_c1_wpl, p1_c1_wpr, p1_c1_bpr, p1_c1_wnl, p1_c1_wnr, p1_c1_bnr,
           p2_c0_wpl, p2_c0_wpr, p2_c0_bpr, p2_c0_wnl, p2_c0_wnr, p2_c0_bnr,
           p2_c1_wpl, p2_c1_wpr, p2_c1_bpr, p2_c1_wnl, p2_c1_wnr, p2_c1_bnr):
    f32 = jnp.float32
    eis = [edge_index_g1, edge_index_g2, edge_index_g3_u,
           edge_index_g3_q, edge_index_g4_u, edge_index_g4_q]
    masks = [m.astype(jnp.int32)[None, :] for m in
             (mask_g1, mask_g2, mask_g3_u, mask_g3_q, mask_g4_u, mask_g4_q)]

    w8 = jnp.stack([W[:, h * H:(h + 1) * H]
                    for W in (Wg1, Wg2, Wg3, Wg4) for h in (0, 1)])

    p1c0 = (p1_c0_wpl, p1_c0_wnl, p1_c0_wpr, p1_c0_wnr, p1_c0_bpr, p1_c0_bnr)
    p2c0 = (p2_c0_wpl, p2_c0_wnl, p2_c0_wpr, p2_c0_wnr, p2_c0_bpr, p2_c0_bnr)
    p1c1 = (p1_c1_wpl, p1_c1_wnl, p1_c1_wpr, p1_c1_wnr, p1_c1_bpr, p1_c1_bnr)
    p2c1 = (p2_c1_wpl, p2_c1_wnl, p2_c1_wpr, p2_c1_wnr, p2_c1_bpr, p2_c1_bnr)

    def block_stacks(pA, pB, pC):
        ws = [jnp.stack([pA[i], pB[i], pC[i]]) for i in range(4)]
        bs = [jnp.stack([pA[i], pB[i], pC[i]])[:, None, :] for i in (4, 5)]
        return ws + bs

    l1a = block_stacks(p1c0, p1c0, p2c0)
    l1b = block_stacks(p2c0, p2c0, p2c0)
    l2a = block_stacks(p1c1, p1c1, p2c1)
    l2b = block_stacks(p2c1, p2c1, p2c1)

    xlr = _tc_proj(x.astype(f32), w8.astype(f32))
    g1a, g1b, g2a, g2b, dka, dkb = _tc_indices(eis, masks)

    z2d = jnp.zeros((ROWS_PER_SUB, H), f32)
    z1d = jnp.zeros((CSLICE,), f32)
    tab1 = xlr.reshape(8 * N, H)

    agg1 = _make_agg(True)
    agg2 = _make_agg(False)

    def cnt5(c):
        return c.reshape(NCORE, NB, CNTP)[:, :, :2 * N].reshape(
            NCORE, NB, 2, N, 1)

    part1a, cnta = agg1(tab1, g1a.reshape(-1, CH), dka.reshape(-1, CH),
                        z2d, z1d)
    part1b, cntb = agg1(tab1, g1b.reshape(-1, CH), dkb.reshape(-1, CH),
                        z2d, z1d)
    cnta5, cntb5 = cnt5(cnta), cnt5(cntb)
    zsta = _tc_layer1(part1a, cnta5, xlr, *l1a, pgl=(0, 1, 2))
    zstb = _tc_layer1(part1b, cntb5, xlr, *l1b, pgl=(2, 3, 3))
    part2a, = agg2(zsta.reshape(NB * 2 * N, H), g2a.reshape(-1, CH),
                   dka.reshape(-1, CH), z2d)
    part2b, = agg2(zstb.reshape(NB * 2 * N, H), g2b.reshape(-1, CH),
                   dkb.reshape(-1, CH), z2d)
    outsa = _tc_layer2(part2a, cnta5, zsta, *l2a)
    outsb = _tc_layer2(part2b, cntb5, zstb, *l2b)

    e1, e2 = outsa[0], outsa[1]
    e3 = jnp.concatenate([outsa[2][:6000], outsb[0][6000:]], axis=0)
    e4 = jnp.concatenate([outsb[1][:6000], outsb[2][6000:]], axis=0)
    return (e1, e2, e3, e4)


# speedup vs baseline: 13.6657x; 1.1880x over previous
import jax
import jax.numpy as jnp
from jax import lax
from jax.experimental import pallas as pl
from jax.experimental.pallas import tpu as pltpu
from jax.experimental.pallas import tpu_sc as plsc

N = 10000
H = 64
E = 320000
NV = 6
NB = 2
CH = 125
NSLOT = 4
NSTG = 8
NSTG_CH = 20
NCORE = 2
NSUB = 16
NCHUNK = 160
CPV = E // CH
ACCR = 20096
ROWS_PER_SUB = ACCR // NSUB
CNTP = 20480
CSLICE = CNTP // NSUB
PGL = (0, 1, 2, 2, 3, 3)


def _tc_proj(x, w8):
    def body(x_ref, w_ref, o_ref):
        o_ref[0] = lax.dot(x_ref[...], w_ref[0],
                           preferred_element_type=jnp.float32)
    return pl.pallas_call(
        body,
        grid=(8,),
        in_specs=[
            pl.BlockSpec((N, 128), lambda g: (0, 0)),
            pl.BlockSpec((1, 128, H), lambda g: (g, 0, 0)),
        ],
        out_specs=pl.BlockSpec((1, N, H), lambda g: (g, 0, 0)),
        out_shape=jax.ShapeDtypeStruct((8, N, H), jnp.float32),
    )(x, w8)


def _tc_indices(eis, masks):
    B = 32000

    def body(*refs):
        ei_refs = refs[0:NV]
        m_refs = refs[NV:2 * NV]
        g1s = refs[2 * NV:2 * NV + 3]
        g2s = refs[2 * NV + 3:2 * NV + 6]
        dks = refs[2 * NV + 6:2 * NV + 9]
        for v in range(NV):
            blk, vl = divmod(v, NB)
            g1, g2, dk = g1s[blk], g2s[blk], dks[blk]
            src = ei_refs[v][0]
            dst = ei_refs[v][1]
            m = m_refs[v][0]
            dk[vl] = jnp.where(m == 1, dst, dst + N)
            for h in range(2):
                g1[h, vl] = src + (2 * PGL[v] + h) * N
                g2[h, vl] = src + (2 * vl + h) * N

    nb = E // B
    io = jax.ShapeDtypeStruct((2, NB, E), jnp.int32)
    do = jax.ShapeDtypeStruct((NB, E), jnp.int32)
    return pl.pallas_call(
        body,
        grid=(nb,),
        in_specs=[pl.BlockSpec((2, B), lambda i: (0, i))] * NV
        + [pl.BlockSpec((1, B), lambda i: (0, i))] * NV,
        out_specs=[pl.BlockSpec((2, NB, B), lambda i: (0, 0, i))] * 6
        + [pl.BlockSpec((NB, B), lambda i: (0, i))] * 3,
        out_shape=[io] * 6 + [do] * 3,
    )(*eis, *masks)


def _mk_pg(pgl):
    def f(v):
        r = pgl[0]
        for i in range(1, len(pgl)):
            r = r + (pgl[i] - pgl[i - 1]) * (v >= i)
        return r
    return f


def _tc_layer1(part, cnt, xlr, wpl, wnl, wpr, wnr, bp, bn, pgl):
    R = 1000
    pg = _mk_pg(pgl)

    def body(pp, pn, cp_r, cn_r, xl_r, xr_r, wpl_r, wnl_r, wpr_r, wnr_r,
             bp_r, bn_r, z_ref):
        dot = lambda a, b: lax.dot(a, b, preferred_element_type=jnp.float32)
        rp = 1.0 / jnp.maximum(cp_r[0, 0], 1.0)
        rn = 1.0 / jnp.maximum(cn_r[0, 0], 1.0)
        mpL = pp[0, 0] * rp
        mpR = pp[0, 1] * rp
        mnL = pn[0, 0] * rn
        mnR = pn[0, 1] * rn
        XL, XR = xl_r[0], xr_r[0]
        wl_p, wl_n, wr_p, wr_n = wpl_r[0], wnl_r[0], wpr_r[0], wnr_r[0]
        zL = (dot(mpL, wl_p[:H]) + dot(mpR, wl_p[H:])
              + dot(XL, wr_p[:H]) + dot(XR, wr_p[H:]) + bp_r[0, 0])
        zR = (dot(mnL, wl_n[:H]) + dot(mnR, wl_n[H:])
              + dot(XL, wr_n[:H]) + dot(XR, wr_n[H:]) + bn_r[0, 0])
        z_ref[0, 0] = jnp.maximum(zL, 0.0)
        z_ref[0, 1] = jnp.maximum(zR, 0.0)

    return pl.pallas_call(
        body,
        grid=(NB, N // R),
        in_specs=[
            pl.BlockSpec((1, 2, R, H), lambda v, r: (v, 0, r, 0)),
            pl.BlockSpec((1, 2, R, H), lambda v, r: (v, 0, 10 + r, 0)),
            pl.BlockSpec((1, 1, R, 1), lambda v, r: (v, 0, r, 0)),
            pl.BlockSpec((1, 1, R, 1), lambda v, r: (v, 1, r, 0)),
            pl.BlockSpec((1, R, H), lambda v, r: (2 * pg(v), r, 0)),
            pl.BlockSpec((1, R, H), lambda v, r: (2 * pg(v) + 1, r, 0)),
            pl.BlockSpec((1, 128, H), lambda v, r: (v, 0, 0)),
            pl.BlockSpec((1, 128, H), lambda v, r: (v, 0, 0)),
            pl.BlockSpec((1, 128, H), lambda v, r: (v, 0, 0)),
            pl.BlockSpec((1, 128, H), lambda v, r: (v, 0, 0)),
            pl.BlockSpec((1, 1, H), lambda v, r: (v, 0, 0)),
            pl.BlockSpec((1, 1, H), lambda v, r: (v, 0, 0)),
        ],
        out_specs=pl.BlockSpec((1, 2, R, H), lambda v, r: (v, 0, r, 0)),
        out_shape=jax.ShapeDtypeStruct((NB, 2, N, H), jnp.float32),
    )(part, part, cnt, cnt, xlr, xlr, wpl, wnl, wpr, wnr, bp, bn)


def _tc_layer2(part, cnt, zst, wpl, wnl, wpr, wnr, bp, bn):
    R = 1000

    def body(pp, pn, cp_r, cn_r, zl_r, zr_r, wpl_r, wnl_r, wpr_r, wnr_r,
             bp_r, bn_r, o_ref):
        dot = lambda a, b: lax.dot(a, b, preferred_element_type=jnp.float32)
        rp = 1.0 / jnp.maximum(cp_r[0, 0], 1.0)
        rn = 1.0 / jnp.maximum(cn_r[0, 0], 1.0)
        MpL = pp[0, 0] * rp
        MpR = pp[0, 1] * rp
        MnL = pn[0, 0] * rn
        MnR = pn[0, 1] * rn
        zL, zR = zl_r[0, 0], zr_r[0, 0]
        wl_p, wl_n, wr_p, wr_n = wpl_r[0], wnl_r[0], wpr_r[0], wnr_r[0]
        op = dot(MpL, wl_p[:H]) + dot(MnR, wl_p[H:]) + dot(zL, wr_p) + bp_r[0, 0]
        on = dot(MpR, wl_n[:H]) + dot(MnL, wl_n[H:]) + dot(zR, wr_n) + bn_r[0, 0]
        o_ref[0] = jnp.maximum(jnp.concatenate([op, on], axis=1), 0.0)

    return pl.pallas_call(
        body,
        grid=(NB, N // R),
        in_specs=[
            pl.BlockSpec((1, 2, R, H), lambda v, r: (v, 0, r, 0)),
            pl.BlockSpec((1, 2, R, H), lambda v, r: (v, 0, 10 + r, 0)),
            pl.BlockSpec((1, 1, R, 1), lambda v, r: (v, 0, r, 0)),
            pl.BlockSpec((1, 1, R, 1), lambda v, r: (v, 1, r, 0)),
            pl.BlockSpec((1, 1, R, H), lambda v, r: (v, 0, r, 0)),
            pl.BlockSpec((1, 1, R, H), lambda v, r: (v, 1, r, 0)),
            pl.BlockSpec((1, 128, H), lambda v, r: (v, 0, 0)),
            pl.BlockSpec((1, 128, H), lambda v, r: (v, 0, 0)),
            pl.BlockSpec((1, H, H), lambda v, r: (v, 0, 0)),
            pl.BlockSpec((1, H, H), lambda v, r: (v, 0, 0)),
            pl.BlockSpec((1, 1, H), lambda v, r: (v, 0, 0)),
            pl.BlockSpec((1, 1, H), lambda v, r: (v, 0, 0)),
        ],
        out_specs=pl.BlockSpec((1, R, 2 * H), lambda v, r: (v, r, 0)),
        out_shape=jax.ShapeDtypeStruct((NB, N, 2 * H), jnp.float32),
    )(part, part, cnt, cnt, zst, zst, wpl, wnl, wpr, wnr, bp, bn)


def _make_agg(do_count):
    mesh = plsc.VectorSubcoreMesh(core_axis_name="c", subcore_axis_name="s")
    out_type = [jax.ShapeDtypeStruct((NB, 2, ACCR, H), jnp.float32)]
    scratch = [pltpu.VMEM((NSTG_CH, CH), jnp.int32) for _ in range(4)]
    scratch += [pltpu.VMEM((CH, H), jnp.float32) for _ in range(NSLOT)]
    scratch += [pltpu.VMEM_SHARED((ACCR, H), jnp.float32)]
    scratch += [pltpu.SemaphoreType.DMA for _ in range(2 * NSLOT)]
    if do_count:
        out_type.append(jax.ShapeDtypeStruct(
            (NB, NSUB, 1, CSLICE), jnp.float32))
        scratch += [
            pltpu.VMEM((128,), jnp.float32),
            pltpu.VMEM_SHARED((CNTP,), jnp.float32),
        ]
        scratch += [pltpu.SemaphoreType.DMA for _ in range(NSLOT)]

    def body(tab, gidx, dstk, *rest):
        if do_count:
            (z2d, z1d, part, cnt, ga, gb, da, db, r0, r1, r2, r3, acc,
             g0, g1, g2, g3, s0, s1, s2, s3,
             ones, cntacc, c0, c1, c2, c3) = rest
            scs = [c0, c1, c2, c3]
        else:
            (z2d, part, ga, gb, da, db, r0, r1, r2, r3, acc,
             g0, g1, g2, g3, s0, s1, s2, s3) = rest
        rows = [r0, r1, r2, r3]
        sgs = [g0, g1, g2, g3]
        sss = [s0, s1, s2, s3]
        cid = lax.axis_index("c")
        sid = lax.axis_index("s")

        if do_count:
            one16 = jnp.ones((16,), jnp.float32)
            for q in range(8):
                ones[pl.ds(q * 16, 16)] = one16

        def do_pass(h, carry):

            pltpu.sync_copy(z2d, acc.at[pl.ds(sid * ROWS_PER_SUB,
                                              ROWS_PER_SUB)])
            if do_count:
                @pl.when(h == 0)
                def _():
                    pltpu.sync_copy(z1d,
                                    cntacc.at[pl.ds(sid * CSLICE, CSLICE)])
            plsc.subcore_barrier()

            gchunk = (h * NB + cid) * CPV + sid * NCHUNK
            dchunk = cid * CPV + sid * NCHUNK

            def issue_scatter(m, s, qd):
                pltpu.async_copy(rows[s], acc.at[qd.at[m]], sss[s], add=True)
                if do_count:
                    @pl.when(h == 0)
                    def _():
                        pltpu.async_copy(ones.at[pl.ds(0, CH)],
                                         cntacc.at[qd.at[m]],
                                         scs[s], add=True)

            def wait_scatter(s, qd):
                pltpu.make_async_copy(rows[s], acc.at[qd.at[0]],
                                      sss[s]).wait()
                if do_count:
                    @pl.when(h == 0)
                    def _():
                        pltpu.make_async_copy(ones.at[pl.ds(0, CH)],
                                              cntacc.at[qd.at[0]],
                                              scs[s]).wait()

            def wait_gather(s, qg):
                pltpu.make_async_copy(tab.at[qg.at[0]], rows[s],
                                      sgs[s]).wait()

            for s4 in range(NSTG):
                qg = (ga, gb)[s4 % 2]
                qd = (da, db)[s4 % 2]
                pltpu.sync_copy(
                    gidx.at[pl.ds(gchunk + s4 * NSTG_CH, NSTG_CH)], qg)
                pltpu.sync_copy(
                    dstk.at[pl.ds(dchunk + s4 * NSTG_CH, NSTG_CH)], qd)

                def quad(qq, c, s4=s4, qg=qg, qd=qd):
                    for s in range(NSLOT):
                        m = qq * NSLOT + s
                        jg = s4 * NSTG_CH + m

                        @pl.when(jg >= NSLOT)
                        def _(s=s, qd=qd):
                            wait_scatter(s, qd)
                        pltpu.async_copy(tab.at[qg.at[m]], rows[s], sgs[s])

                        @pl.when(m >= 1)
                        def _(m=m, s=s, qg=qg, qd=qd):
                            wait_gather((s - 1) % NSLOT, qg)
                            issue_scatter(m - 1, (s - 1) % NSLOT, qd)
                    return c
                lax.fori_loop(0, NSTG_CH // NSLOT, quad, 0)
                wait_gather(NSLOT - 1, qg)
                issue_scatter(NSTG_CH - 1, NSLOT - 1, qd)

            for s in range(NSLOT):
                wait_scatter(s, db)
            plsc.subcore_barrier()

            rbase = sid * ROWS_PER_SUB
            pltpu.sync_copy(
                acc.at[pl.ds(rbase, ROWS_PER_SUB)],
                part.at[cid, h, pl.ds(rbase, ROWS_PER_SUB)])
            if do_count:
                @pl.when(h == 0)
                def _():
                    pltpu.sync_copy(
                        cntacc.at[pl.ds(sid * CSLICE, CSLICE)],
                        cnt.at[cid, sid, 0])
            return carry

        lax.fori_loop(0, 2, do_pass, 0)

    return pl.kernel(
        body, out_type=out_type, mesh=mesh, scratch_types=scratch,
        compiler_params=pltpu.CompilerParams(use_tc_tiling_on_sc=False))


def kernel(x, edge_index_g1, mask_g1, edge_index_g2, mask_g2,
           edge_index_g3_u, mask_g3_u, edge_index_g3_q, mask_g3_q,
           edge_index_g4_u, mask_g4_u, edge_index_g4_q, mask_g4_q,
           Wg1, Wg2, Wg3, Wg4,
           p1_c0_wpl, p1_c0_wpr, p1_c0_bpr, p1_c0_wnl, p1_c0_wnr, p1_c0_bnr,
           p1_c1_wpl, p1_c1_wpr, p1_c1_bpr, p1_c1_wnl, p1_c1_wnr, p1_c1_bnr,
           p2_c0_wpl, p2_c0_wpr, p2_c0_bpr, p2_c0_wnl, p2_c0_wnr, p2_c0_bnr,
           p2_c1_wpl, p2_c1_wpr, p2_c1_bpr, p2_c1_wnl, p2_c1_wnr, p2_c1_bnr):
    f32 = jnp.float32
    eis = [edge_index_g1, edge_index_g2, edge_index_g3_u,
           edge_index_g3_q, edge_index_g4_u, edge_index_g4_q]
    masks = [m.astype(jnp.int32)[None, :] for m in
             (mask_g1, mask_g2, mask_g3_u, mask_g3_q, mask_g4_u, mask_g4_q)]

    w8 = jnp.stack([W[:, h * H:(h + 1) * H]
                    for W in (Wg1, Wg2, Wg3, Wg4) for h in (0, 1)])

    p1c0 = (p1_c0_wpl, p1_c0_wnl, p1_c0_wpr, p1_c0_wnr, p1_c0_bpr, p1_c0_bnr)
    p2c0 = (p2_c0_wpl, p2_c0_wnl, p2_c0_wpr, p2_c0_wnr, p2_c0_bpr, p2_c0_bnr)
    p1c1 = (p1_c1_wpl, p1_c1_wnl, p1_c1_wpr, p1_c1_wnr, p1_c1_bpr, p1_c1_bnr)
    p2c1 = (p2_c1_wpl, p2_c1_wnl, p2_c1_wpr, p2_c1_wnr, p2_c1_bpr, p2_c1_bnr)

    def block_stacks(pA, pB):
        ws = [jnp.stack([pA[i], pB[i]]) for i in range(4)]
        bs = [jnp.stack([pA[i], pB[i]])[:, None, :] for i in (4, 5)]
        return ws + bs

    l1 = [block_stacks(p1c0, p1c0), block_stacks(p2c0, p2c0),
          block_stacks(p2c0, p2c0)]
    l2 = [block_stacks(p1c1, p1c1), block_stacks(p2c1, p2c1),
          block_stacks(p2c1, p2c1)]
    pgls = [(0, 1), (2, 2), (3, 3)]

    xlr = _tc_proj(x.astype(f32), w8.astype(f32))
    idx_out = _tc_indices(eis, masks)
    g1s, g2s, dks = idx_out[0:3], idx_out[3:6], idx_out[6:9]

    z2d = jnp.zeros((ROWS_PER_SUB, H), f32)
    z1d = jnp.zeros((CSLICE,), f32)
    tab1 = xlr.reshape(8 * N, H)

    agg1 = _make_agg(True)
    agg2 = _make_agg(False)

    def cnt5(c):
        return c.reshape(NB, CNTP)[:, :2 * N].reshape(NB, 2, N, 1)

    parts1, cnts = [], []
    for blk in range(3):
        p_, c_ = agg1(tab1, g1s[blk].reshape(-1, CH),
                      dks[blk].reshape(-1, CH), z2d, z1d)
        parts1.append(p_)
        cnts.append(cnt5(c_))
    zsts = [_tc_layer1(parts1[blk], cnts[blk], xlr, *l1[blk], pgl=pgls[blk])
            for blk in range(3)]
    parts2 = [agg2(zsts[blk].reshape(NB * 2 * N, H),
                   g2s[blk].reshape(-1, CH),
                   dks[blk].reshape(-1, CH), z2d)[0]
              for blk in range(3)]
    outs = [_tc_layer2(parts2[blk], cnts[blk], zsts[blk], *l2[blk])
            for blk in range(3)]

    e1, e2 = outs[0][0], outs[0][1]
    e3 = jnp.concatenate([outs[1][0][:6000], outs[1][1][6000:]], axis=0)
    e4 = jnp.concatenate([outs[2][0][:6000], outs[2][1][6000:]], axis=0)
    return (e1, e2, e3, e4)


# speedup vs baseline: 13.8678x; 1.0148x over previous
import jax
import jax.numpy as jnp
from jax import lax
from jax.experimental import pallas as pl
from jax.experimental.pallas import tpu as pltpu
from jax.experimental.pallas import tpu_sc as plsc

N = 10000
H = 64
E = 320000
NV = 6
NB = 2
CH = 125
NSLOT = 4
NSTG = 8
NSTG_CH = 20
NCORE = 2
NSUB = 16
NCHUNK = 160
CPV = E // CH
ACCR = 20096
ROWS_PER_SUB = ACCR // NSUB
CNTP = 20480
CSLICE = CNTP // NSUB
PGL = (0, 1, 2, 2, 3, 3)


def _tc_proj(x, w8):
    def body(x_ref, w_ref, o_ref):
        o_ref[0] = lax.dot(x_ref[...], w_ref[0],
                           preferred_element_type=jnp.float32)
    return pl.pallas_call(
        body,
        grid=(8,),
        in_specs=[
            pl.BlockSpec((N, 128), lambda g: (0, 0)),
            pl.BlockSpec((1, 128, H), lambda g: (g, 0, 0)),
        ],
        out_specs=pl.BlockSpec((1, N, H), lambda g: (g, 0, 0)),
        out_shape=jax.ShapeDtypeStruct((8, N, H), jnp.float32),
    )(x, w8)


def _tc_indices(eis2, masks2, vbase):
    B = 32000

    def body(*refs):
        ei_refs = refs[0:NB]
        m_refs = refs[NB:2 * NB]
        g1, g2, dk = refs[2 * NB:]
        for vl in range(NB):
            src = ei_refs[vl][0]
            dst = ei_refs[vl][1]
            m = m_refs[vl][0]
            dk[vl] = jnp.where(m == 1, dst, dst + N)
            for h in range(2):
                g1[h, vl] = src + (2 * PGL[vbase + vl] + h) * N
                g2[h, vl] = src + (2 * vl + h) * N

    nb = E // B
    io = jax.ShapeDtypeStruct((2, NB, E), jnp.int32)
    do = jax.ShapeDtypeStruct((NB, E), jnp.int32)
    return pl.pallas_call(
        body,
        grid=(nb,),
        in_specs=[pl.BlockSpec((2, B), lambda i: (0, i))] * NB
        + [pl.BlockSpec((1, B), lambda i: (0, i))] * NB,
        out_specs=[pl.BlockSpec((2, NB, B), lambda i: (0, 0, i))] * 2
        + [pl.BlockSpec((NB, B), lambda i: (0, i))],
        out_shape=[io, io, do],
    )(*eis2, *masks2)


def _mk_pg(pgl):
    def f(v):
        r = pgl[0]
        for i in range(1, len(pgl)):
            r = r + (pgl[i] - pgl[i - 1]) * (v >= i)
        return r
    return f


def _tc_layer1(part, cnt, xlr, wpl, wnl, wpr, wnr, bp, bn, pgl):
    R = 1000
    pg = _mk_pg(pgl)

    def body(pp, pn, cp_r, cn_r, xl_r, xr_r, wpl_r, wnl_r, wpr_r, wnr_r,
             bp_r, bn_r, z_ref):
        dot = lambda a, b: lax.dot(a, b, preferred_element_type=jnp.float32)
        rp = 1.0 / jnp.maximum(cp_r[0, 0], 1.0)
        rn = 1.0 / jnp.maximum(cn_r[0, 0], 1.0)
        mpL = pp[0, 0] * rp
        mpR = pp[0, 1] * rp
        mnL = pn[0, 0] * rn
        mnR = pn[0, 1] * rn
        XL, XR = xl_r[0], xr_r[0]
        wl_p, wl_n, wr_p, wr_n = wpl_r[0], wnl_r[0], wpr_r[0], wnr_r[0]
        zL = (dot(mpL, wl_p[:H]) + dot(mpR, wl_p[H:])
              + dot(XL, wr_p[:H]) + dot(XR, wr_p[H:]) + bp_r[0, 0])
        zR = (dot(mnL, wl_n[:H]) + dot(mnR, wl_n[H:])
              + dot(XL, wr_n[:H]) + dot(XR, wr_n[H:]) + bn_r[0, 0])
        z_ref[0, 0] = jnp.maximum(zL, 0.0)
        z_ref[0, 1] = jnp.maximum(zR, 0.0)

    return pl.pallas_call(
        body,
        grid=(NB, N // R),
        in_specs=[
            pl.BlockSpec((1, 2, R, H), lambda v, r: (v, 0, r, 0)),
            pl.BlockSpec((1, 2, R, H), lambda v, r: (v, 0, 10 + r, 0)),
            pl.BlockSpec((1, 1, R, 1), lambda v, r: (v, 0, r, 0)),
            pl.BlockSpec((1, 1, R, 1), lambda v, r: (v, 1, r, 0)),
            pl.BlockSpec((1, R, H), lambda v, r: (2 * pg(v), r, 0)),
            pl.BlockSpec((1, R, H), lambda v, r: (2 * pg(v) + 1, r, 0)),
            pl.BlockSpec((1, 128, H), lambda v, r: (v, 0, 0)),
            pl.BlockSpec((1, 128, H), lambda v, r: (v, 0, 0)),
            pl.BlockSpec((1, 128, H), lambda v, r: (v, 0, 0)),
            pl.BlockSpec((1, 128, H), lambda v, r: (v, 0, 0)),
            pl.BlockSpec((1, 1, H), lambda v, r: (v, 0, 0)),
            pl.BlockSpec((1, 1, H), lambda v, r: (v, 0, 0)),
        ],
        out_specs=pl.BlockSpec((1, 2, R, H), lambda v, r: (v, 0, r, 0)),
        out_shape=jax.ShapeDtypeStruct((NB, 2, N, H), jnp.float32),
    )(part, part, cnt, cnt, xlr, xlr, wpl, wnl, wpr, wnr, bp, bn)


def _tc_layer2(part, cnt, zst, wpl, wnl, wpr, wnr, bp, bn):
    R = 1000

    def body(pp, pn, cp_r, cn_r, zl_r, zr_r, wpl_r, wnl_r, wpr_r, wnr_r,
             bp_r, bn_r, o_ref):
        dot = lambda a, b: lax.dot(a, b, preferred_element_type=jnp.float32)
        rp = 1.0 / jnp.maximum(cp_r[0, 0], 1.0)
        rn = 1.0 / jnp.maximum(cn_r[0, 0], 1.0)
        MpL = pp[0, 0] * rp
        MpR = pp[0, 1] * rp
        MnL = pn[0, 0] * rn
        MnR = pn[0, 1] * rn
        zL, zR = zl_r[0, 0], zr_r[0, 0]
        wl_p, wl_n, wr_p, wr_n = wpl_r[0], wnl_r[0], wpr_r[0], wnr_r[0]
        op = dot(MpL, wl_p[:H]) + dot(MnR, wl_p[H:]) + dot(zL, wr_p) + bp_r[0, 0]
        on = dot(MpR, wl_n[:H]) + dot(MnL, wl_n[H:]) + dot(zR, wr_n) + bn_r[0, 0]
        o_ref[0] = jnp.maximum(jnp.concatenate([op, on], axis=1), 0.0)

    return pl.pallas_call(
        body,
        grid=(NB, N // R),
        in_specs=[
            pl.BlockSpec((1, 2, R, H), lambda v, r: (v, 0, r, 0)),
            pl.BlockSpec((1, 2, R, H), lambda v, r: (v, 0, 10 + r, 0)),
            pl.BlockSpec((1, 1, R, 1), lambda v, r: (v, 0, r, 0)),
            pl.BlockSpec((1, 1, R, 1), lambda v, r: (v, 1, r, 0)),
            pl.BlockSpec((1, 1, R, H), lambda v, r: (v, 0, r, 0)),
            pl.BlockSpec((1, 1, R, H), lambda v, r: (v, 1, r, 0)),
            pl.BlockSpec((1, 128, H), lambda v, r: (v, 0, 0)),
            pl.BlockSpec((1, 128, H), lambda v, r: (v, 0, 0)),
            pl.BlockSpec((1, H, H), lambda v, r: (v, 0, 0)),
            pl.BlockSpec((1, H, H), lambda v, r: (v, 0, 0)),
            pl.BlockSpec((1, 1, H), lambda v, r: (v, 0, 0)),
            pl.BlockSpec((1, 1, H), lambda v, r: (v, 0, 0)),
        ],
        out_specs=pl.BlockSpec((1, R, 2 * H), lambda v, r: (v, r, 0)),
        out_shape=jax.ShapeDtypeStruct((NB, N, 2 * H), jnp.float32),
    )(part, part, cnt, cnt, zst, zst, wpl, wnl, wpr, wnr, bp, bn)


def _make_agg(do_count):
    mesh = plsc.VectorSubcoreMesh(core_axis_name="c", subcore_axis_name="s")
    out_type = [jax.ShapeDtypeStruct((NB, 2, ACCR, H), jnp.float32)]
    scratch = [pltpu.VMEM((NSTG_CH, CH), jnp.int32) for _ in range(4)]
    scratch += [pltpu.VMEM((CH, H), jnp.float32) for _ in range(NSLOT)]
    scratch += [pltpu.VMEM_SHARED((ACCR, H), jnp.float32)]
    scratch += [pltpu.SemaphoreType.DMA for _ in range(2 * NSLOT)]
    if do_count:
        out_type.append(jax.ShapeDtypeStruct(
            (NB, NSUB, 1, CSLICE), jnp.float32))
        scratch += [
            pltpu.VMEM((128,), jnp.float32),
            pltpu.VMEM_SHARED((CNTP,), jnp.float32),
        ]
        scratch += [pltpu.SemaphoreType.DMA for _ in range(NSLOT)]

    def body(tab, gidx, dstk, *rest):
        if do_count:
            (z2d, z1d, part, cnt, ga, gb, da, db, r0, r1, r2, r3, acc,
             g0, g1, g2, g3, s0, s1, s2, s3,
             ones, cntacc, c0, c1, c2, c3) = rest
            scs = [c0, c1, c2, c3]
        else:
            (z2d, part, ga, gb, da, db, r0, r1, r2, r3, acc,
             g0, g1, g2, g3, s0, s1, s2, s3) = rest
        rows = [r0, r1, r2, r3]
        sgs = [g0, g1, g2, g3]
        sss = [s0, s1, s2, s3]
        cid = lax.axis_index("c")
        sid = lax.axis_index("s")

        if do_count:
            one16 = jnp.ones((16,), jnp.float32)
            for q in range(8):
                ones[pl.ds(q * 16, 16)] = one16

        def do_pass(h, carry):

            pltpu.sync_copy(z2d, acc.at[pl.ds(sid * ROWS_PER_SUB,
                                              ROWS_PER_SUB)])
            if do_count:
                @pl.when(h == 0)
                def _():
                    pltpu.sync_copy(z1d,
                                    cntacc.at[pl.ds(sid * CSLICE, CSLICE)])
            plsc.subcore_barrier()

            gchunk = (h * NB + cid) * CPV + sid * NCHUNK
            dchunk = cid * CPV + sid * NCHUNK

            def issue_scatter(m, s, qd):
                pltpu.async_copy(rows[s], acc.at[qd.at[m]], sss[s], add=True)
                if do_count:
                    @pl.when(h == 0)
                    def _():
                        pltpu.async_copy(ones.at[pl.ds(0, CH)],
                                         cntacc.at[qd.at[m]],
                                         scs[s], add=True)

            def wait_scatter(s, qd):
                pltpu.make_async_copy(rows[s], acc.at[qd.at[0]],
                                      sss[s]).wait()
                if do_count:
                    @pl.when(h == 0)
                    def _():
                        pltpu.make_async_copy(ones.at[pl.ds(0, CH)],
                                              cntacc.at[qd.at[0]],
                                              scs[s]).wait()

            def wait_gather(s, qg):
                pltpu.make_async_copy(tab.at[qg.at[0]], rows[s],
                                      sgs[s]).wait()

            for s4 in range(NSTG):
                qg = (ga, gb)[s4 % 2]
                qd = (da, db)[s4 % 2]
                pltpu.sync_copy(
                    gidx.at[pl.ds(gchunk + s4 * NSTG_CH, NSTG_CH)], qg)
                pltpu.sync_copy(
                    dstk.at[pl.ds(dchunk + s4 * NSTG_CH, NSTG_CH)], qd)

                def quad(qq, c, s4=s4, qg=qg, qd=qd):
                    for s in range(NSLOT):
                        m = qq * NSLOT + s
                        jg = s4 * NSTG_CH + m

                        @pl.when(jg >= NSLOT)
                        def _(s=s, qd=qd):
                            wait_scatter(s, qd)
                        pltpu.async_copy(tab.at[qg.at[m]], rows[s], sgs[s])

                        @pl.when(m >= 1)
                        def _(m=m, s=s, qg=qg, qd=qd):
                            wait_gather((s - 1) % NSLOT, qg)
                            issue_scatter(m - 1, (s - 1) % NSLOT, qd)
                    return c
                lax.fori_loop(0, NSTG_CH // NSLOT, quad, 0)
                wait_gather(NSLOT - 1, qg)
                issue_scatter(NSTG_CH - 1, NSLOT - 1, qd)

            for s in range(NSLOT):
                wait_scatter(s, db)
            plsc.subcore_barrier()

            rbase = sid * ROWS_PER_SUB
            pltpu.sync_copy(
                acc.at[pl.ds(rbase, ROWS_PER_SUB)],
                part.at[cid, h, pl.ds(rbase, ROWS_PER_SUB)])
            if do_count:
                @pl.when(h == 0)
                def _():
                    pltpu.sync_copy(
                        cntacc.at[pl.ds(sid * CSLICE, CSLICE)],
                        cnt.at[cid, sid, 0])
            return carry

        lax.fori_loop(0, 2, do_pass, 0)

    return pl.kernel(
        body, out_type=out_type, mesh=mesh, scratch_types=scratch,
        compiler_params=pltpu.CompilerParams(use_tc_tiling_on_sc=False))


def kernel(x, edge_index_g1, mask_g1, edge_index_g2, mask_g2,
           edge_index_g3_u, mask_g3_u, edge_index_g3_q, mask_g3_q,
           edge_index_g4_u, mask_g4_u, edge_index_g4_q, mask_g4_q,
           Wg1, Wg2, Wg3, Wg4,
           p1_c0_wpl, p1_c0_wpr, p1_c0_bpr, p1_c0_wnl, p1_c0_wnr, p1_c0_bnr,
           p1_c1_wpl, p1_c1_wpr, p1_c1_bpr, p1_c1_wnl, p1_c1_wnr, p1_c1_bnr,
           p2_c0_wpl, p2_c0_wpr, p2_c0_bpr, p2_c0_wnl, p2_c0_wnr, p2_c0_bnr,
           p2_c1_wpl, p2_c1_wpr, p2_c1_bpr, p2_c1_wnl, p2_c1_wnr, p2_c1_bnr):
    f32 = jnp.float32
    eis = [edge_index_g1, edge_index_g2, edge_index_g3_u,
           edge_index_g3_q, edge_index_g4_u, edge_index_g4_q]
    masks = [m.astype(jnp.int32)[None, :] for m in
             (mask_g1, mask_g2, mask_g3_u, mask_g3_q, mask_g4_u, mask_g4_q)]

    w8 = jnp.stack([W[:, h * H:(h + 1) * H]
                    for W in (Wg1, Wg2, Wg3, Wg4) for h in (0, 1)])

    p1c0 = (p1_c0_wpl, p1_c0_wnl, p1_c0_wpr, p1_c0_wnr, p1_c0_bpr, p1_c0_bnr)
    p2c0 = (p2_c0_wpl, p2_c0_wnl, p2_c0_wpr, p2_c0_wnr, p2_c0_bpr, p2_c0_bnr)
    p1c1 = (p1_c1_wpl, p1_c1_wnl, p1_c1_wpr, p1_c1_wnr, p1_c1_bpr, p1_c1_bnr)
    p2c1 = (p2_c1_wpl, p2_c1_wnl, p2_c1_wpr, p2_c1_wnr, p2_c1_bpr, p2_c1_bnr)

    def block_stacks(pA, pB):
        ws = [jnp.stack([pA[i], pB[i]]) for i in range(4)]
        bs = [jnp.stack([pA[i], pB[i]])[:, None, :] for i in (4, 5)]
        return ws + bs

    l1 = [block_stacks(p1c0, p1c0), block_stacks(p2c0, p2c0),
          block_stacks(p2c0, p2c0)]
    l2 = [block_stacks(p1c1, p1c1), block_stacks(p2c1, p2c1),
          block_stacks(p2c1, p2c1)]
    pgls = [(0, 1), (2, 2), (3, 3)]

    xlr = _tc_proj(x.astype(f32), w8.astype(f32))
    g1s, g2s, dks = [], [], []
    for blk in range(3):
        g1_, g2_, dk_ = _tc_indices(eis[2 * blk:2 * blk + 2],
                                    masks[2 * blk:2 * blk + 2], 2 * blk)
        g1s.append(g1_)
        g2s.append(g2_)
        dks.append(dk_)

    z2d = jnp.zeros((ROWS_PER_SUB, H), f32)
    z1d = jnp.zeros((CSLICE,), f32)
    tab1 = xlr.reshape(8 * N, H)

    agg1 = _make_agg(True)
    agg2 = _make_agg(False)

    def cnt5(c):
        return c.reshape(NB, CNTP)[:, :2 * N].reshape(NB, 2, N, 1)

    parts1, cnts = [], []
    for blk in range(3):
        p_, c_ = agg1(tab1, g1s[blk].reshape(-1, CH),
                      dks[blk].reshape(-1, CH), z2d, z1d)
        parts1.append(p_)
        cnts.append(cnt5(c_))
    zsts = [_tc_layer1(parts1[blk], cnts[blk], xlr, *l1[blk], pgl=pgls[blk])
            for blk in range(3)]
    parts2 = [agg2(zsts[blk].reshape(NB * 2 * N, H),
                   g2s[blk].reshape(-1, CH),
                   dks[blk].reshape(-1, CH), z2d)[0]
              for blk in range(3)]
    outs = [_tc_layer2(parts2[blk], cnts[blk], zsts[blk], *l2[blk])
            for blk in range(3)]

    e1, e2 = outs[0][0], outs[0][1]
    e3 = jnp.concatenate([outs[1][0][:6000], outs[1][1][6000:]], axis=0)
    e4 = jnp.concatenate([outs[2][0][:6000], outs[2][1][6000:]], axis=0)
    return (e1, e2, e3, e4)


# speedup vs baseline: 14.1513x; 1.0204x over previous
import jax
import jax.numpy as jnp
from jax import lax
from jax.experimental import pallas as pl
from jax.experimental.pallas import tpu as pltpu
from jax.experimental.pallas import tpu_sc as plsc

N = 10000
H = 64
Q = 32
E = 320000
NV = 6
NB = 2
CH = 125
NSLOT = 4
NCORE = 2
NSUB = 16
NCHUNK = 160
CPV = E // CH
ACCR = 20096
ROWS_PER_SUB = ACCR // NSUB
CNTP = 20480
CSLICE = CNTP // NSUB
PGL = (0, 1, 2, 2, 3, 3)


def _tc_proj(x, w16):
    def body(x_ref, w_ref, o_ref):
        o_ref[0] = lax.dot(x_ref[...], w_ref[0],
                           preferred_element_type=jnp.float32)
    return pl.pallas_call(
        body,
        grid=(16,),
        in_specs=[
            pl.BlockSpec((N, 128), lambda g: (0, 0)),
            pl.BlockSpec((1, 128, Q), lambda g: (g, 0, 0)),
        ],
        out_specs=pl.BlockSpec((1, N, Q), lambda g: (g, 0, 0)),
        out_shape=jax.ShapeDtypeStruct((16, N, Q), jnp.float32),
    )(x, w16)


def _tc_indices(eis2, masks2):
    B = 32000

    def body(*refs):
        ei_refs = refs[0:NB]
        m_refs = refs[NB:2 * NB]
        sr, dk = refs[2 * NB:]
        for vl in range(NB):
            sr[vl] = ei_refs[vl][0]
            dst = ei_refs[vl][1]
            m = m_refs[vl][0]
            dk[vl] = jnp.where(m == 1, dst, dst + N)

    nb = E // B
    do = jax.ShapeDtypeStruct((NB, E), jnp.int32)
    return pl.pallas_call(
        body,
        grid=(nb,),
        in_specs=[pl.BlockSpec((2, B), lambda i: (0, i))] * NB
        + [pl.BlockSpec((1, B), lambda i: (0, i))] * NB,
        out_specs=[pl.BlockSpec((NB, B), lambda i: (0, i))] * 2,
        out_shape=[do, do],
    )(*eis2, *masks2)


def _mk_pg(pgl):
    def f(v):
        r = pgl[0]
        for i in range(1, len(pgl)):
            r = r + (pgl[i] - pgl[i - 1]) * (v >= i)
        return r
    return f


def _tc_layer1(part, cnt, xq, wpl, wnl, wpr, wnr, bp, bn, pgl):
    R = 1000
    pg = _mk_pg(pgl)

    def body(pp, pn, cp_r, cn_r, x0, x1, x2, x3, wpl_r, wnl_r, wpr_r, wnr_r,
             bp_r, bn_r, z_ref):
        dot = lambda a, b: lax.dot(a, b, preferred_element_type=jnp.float32)
        rp = 1.0 / jnp.maximum(cp_r[0, 0], 1.0)
        rn = 1.0 / jnp.maximum(cn_r[0, 0], 1.0)
        mq = [pp[0, i] * rp for i in range(4)]
        nq = [pn[0, i] * rn for i in range(4)]
        xs = [x0[0], x1[0], x2[0], x3[0]]
        wl_p, wl_n, wr_p, wr_n = wpl_r[0], wnl_r[0], wpr_r[0], wnr_r[0]
        zL = bp_r[0, 0] + sum(
            dot(mq[i], wl_p[Q * i:Q * (i + 1)]) for i in range(4)) + sum(
            dot(xs[i], wr_p[Q * i:Q * (i + 1)]) for i in range(4))
        zR = bn_r[0, 0] + sum(
            dot(nq[i], wl_n[Q * i:Q * (i + 1)]) for i in range(4)) + sum(
            dot(xs[i], wr_n[Q * i:Q * (i + 1)]) for i in range(4))
        zL = jnp.maximum(zL, 0.0)
        zR = jnp.maximum(zR, 0.0)
        z_ref[0, 0] = zL[:, :Q]
        z_ref[0, 1] = zL[:, Q:]
        z_ref[0, 2] = zR[:, :Q]
        z_ref[0, 3] = zR[:, Q:]

    return pl.pallas_call(
        body,
        grid=(NB, N // R),
        in_specs=[
            pl.BlockSpec((1, 4, R, Q), lambda v, r: (v, 0, r, 0)),
            pl.BlockSpec((1, 4, R, Q), lambda v, r: (v, 0, 10 + r, 0)),
            pl.BlockSpec((1, 1, R, 1), lambda v, r: (v, 0, r, 0)),
            pl.BlockSpec((1, 1, R, 1), lambda v, r: (v, 1, r, 0)),
            pl.BlockSpec((1, R, Q), lambda v, r: (4 * pg(v), r, 0)),
            pl.BlockSpec((1, R, Q), lambda v, r: (4 * pg(v) + 1, r, 0)),
            pl.BlockSpec((1, R, Q), lambda v, r: (4 * pg(v) + 2, r, 0)),
            pl.BlockSpec((1, R, Q), lambda v, r: (4 * pg(v) + 3, r, 0)),
            pl.BlockSpec((1, 128, H), lambda v, r: (v, 0, 0)),
            pl.BlockSpec((1, 128, H), lambda v, r: (v, 0, 0)),
            pl.BlockSpec((1, 128, H), lambda v, r: (v, 0, 0)),
            pl.BlockSpec((1, 128, H), lambda v, r: (v, 0, 0)),
            pl.BlockSpec((1, 1, H), lambda v, r: (v, 0, 0)),
            pl.BlockSpec((1, 1, H), lambda v, r: (v, 0, 0)),
        ],
        out_specs=pl.BlockSpec((1, 4, R, Q), lambda v, r: (v, 0, r, 0)),
        out_shape=jax.ShapeDtypeStruct((NB, 4, N, Q), jnp.float32),
    )(part, part, cnt, cnt, xq, xq, xq, xq, wpl, wnl, wpr, wnr, bp, bn)


def _tc_layer2(part, cnt, zst, wpl, wnl, wpr, wnr, bp, bn):
    R = 1000

    def body(pp, pn, cp_r, cn_r, z0, z1, z2, z3, wpl_r, wnl_r, wpr_r, wnr_r,
             bp_r, bn_r, o_ref):
        dot = lambda a, b: lax.dot(a, b, preferred_element_type=jnp.float32)
        rp = 1.0 / jnp.maximum(cp_r[0, 0], 1.0)
        rn = 1.0 / jnp.maximum(cn_r[0, 0], 1.0)
        Mp = [pp[0, i] * rp for i in range(4)]
        Mn = [pn[0, i] * rn for i in range(4)]
        zq = [z0[0, 0], z1[0, 0], z2[0, 0], z3[0, 0]]
        wl_p, wl_n, wr_p, wr_n = wpl_r[0], wnl_r[0], wpr_r[0], wnr_r[0]
        op = (bp_r[0, 0]
              + dot(Mp[0], wl_p[0:Q]) + dot(Mp[1], wl_p[Q:2 * Q])
              + dot(Mn[2], wl_p[2 * Q:3 * Q]) + dot(Mn[3], wl_p[3 * Q:])
              + dot(zq[0], wr_p[:Q]) + dot(zq[1], wr_p[Q:]))
        on = (bn_r[0, 0]
              + dot(Mp[2], wl_n[0:Q]) + dot(Mp[3], wl_n[Q:2 * Q])
              + dot(Mn[0], wl_n[2 * Q:3 * Q]) + dot(Mn[1], wl_n[3 * Q:])
              + dot(zq[2], wr_n[:Q]) + dot(zq[3], wr_n[Q:]))
        o_ref[0] = jnp.maximum(jnp.concatenate([op, on], axis=1), 0.0)

    return pl.pallas_call(
        body,
        grid=(NB, N // R),
        in_specs=[
            pl.BlockSpec((1, 4, R, Q), lambda v, r: (v, 0, r, 0)),
            pl.BlockSpec((1, 4, R, Q), lambda v, r: (v, 0, 10 + r, 0)),
            pl.BlockSpec((1, 1, R, 1), lambda v, r: (v, 0, r, 0)),
            pl.BlockSpec((1, 1, R, 1), lambda v, r: (v, 1, r, 0)),
            pl.BlockSpec((1, 1, R, Q), lambda v, r: (v, 0, r, 0)),
            pl.BlockSpec((1, 1, R, Q), lambda v, r: (v, 1, r, 0)),
            pl.BlockSpec((1, 1, R, Q), lambda v, r: (v, 2, r, 0)),
            pl.BlockSpec((1, 1, R, Q), lambda v, r: (v, 3, r, 0)),
            pl.BlockSpec((1, 128, H), lambda v, r: (v, 0, 0)),
            pl.BlockSpec((1, 128, H), lambda v, r: (v, 0, 0)),
            pl.BlockSpec((1, H, H), lambda v, r: (v, 0, 0)),
            pl.BlockSpec((1, H, H), lambda v, r: (v, 0, 0)),
            pl.BlockSpec((1, 1, H), lambda v, r: (v, 0, 0)),
            pl.BlockSpec((1, 1, H), lambda v, r: (v, 0, 0)),
        ],
        out_specs=pl.BlockSpec((1, R, 2 * H), lambda v, r: (v, r, 0)),
        out_shape=jax.ShapeDtypeStruct((NB, N, 2 * H), jnp.float32),
    )(part, part, cnt, cnt, zst, zst, zst, zst, wpl, wnl, wpr, wnr, bp, bn)


def _make_agg(tq0, tq1, do_count):
    mesh = plsc.VectorSubcoreMesh(core_axis_name="c", subcore_axis_name="s")
    out_type = [jax.ShapeDtypeStruct((NB, 4, ACCR, Q), jnp.float32)]
    scratch = [
        pltpu.VMEM((NCHUNK, CH), jnp.int32),
        pltpu.VMEM((NCHUNK, CH), jnp.int32),
    ]
    scratch += [pltpu.VMEM((CH, Q), jnp.float32) for _ in range(NSLOT)]
    scratch += [
        pltpu.VMEM_SHARED((N, Q), jnp.float32),
        pltpu.VMEM_SHARED((ACCR, Q), jnp.float32),
    ]
    scratch += [pltpu.SemaphoreType.DMA for _ in range(2 * NSLOT)]
    if do_count:
        out_type.append(jax.ShapeDtypeStruct(
            (NB, NSUB, 1, CSLICE), jnp.float32))
        scratch += [
            pltpu.VMEM((128,), jnp.float32),
            pltpu.VMEM_SHARED((CNTP,), jnp.float32),
        ]
        scratch += [pltpu.SemaphoreType.DMA for _ in range(NSLOT)]

    def body(tab, srcs, dstk, *rest):
        if do_count:
            (z2d, z1d, part, cnt, idxs, idxd, r0, r1, r2, r3, tabS, acc,
             g0, g1, g2, g3, s0, s1, s2, s3,
             ones, cntacc, c0, c1, c2, c3) = rest
            scs = [c0, c1, c2, c3]
        else:
            (z2d, part, idxs, idxd, r0, r1, r2, r3, tabS, acc,
             g0, g1, g2, g3, s0, s1, s2, s3) = rest
        rows = [r0, r1, r2, r3]
        sgs = [g0, g1, g2, g3]
        sss = [s0, s1, s2, s3]
        cid = lax.axis_index("c")
        sid = lax.axis_index("s")
        tq = tq0 + (tq1 - tq0) * cid

        if do_count:
            one16 = jnp.ones((16,), jnp.float32)
            for q in range(8):
                ones[pl.ds(q * 16, 16)] = one16

        ibase = cid * CPV + sid * NCHUNK
        pltpu.sync_copy(srcs.at[pl.ds(ibase, NCHUNK)], idxs)
        pltpu.sync_copy(dstk.at[pl.ds(ibase, NCHUNK)], idxd)

        def do_pass(qp, carry):
            pltpu.sync_copy(z2d, acc.at[pl.ds(sid * ROWS_PER_SUB,
                                              ROWS_PER_SUB)])
            @pl.when(sid == 0)
            def _():
                pltpu.sync_copy(tab.at[pl.ds((tq + qp) * N, N)], tabS)
            if do_count:
                @pl.when(qp == 0)
                def _():
                    pltpu.sync_copy(z1d,
                                    cntacc.at[pl.ds(sid * CSLICE, CSLICE)])
            plsc.subcore_barrier()

            def issue_scatter(m, s):
                pltpu.async_copy(rows[s], acc.at[idxd.at[m]], sss[s],
                                 add=True)
                if do_count:
                    @pl.when(qp == 0)
                    def _():
                        pltpu.async_copy(ones.at[pl.ds(0, CH)],
                                         cntacc.at[idxd.at[m]],
                                         scs[s], add=True)

            def wait_scatter(s):
                pltpu.make_async_copy(rows[s], acc.at[idxd.at[0]],
                                      sss[s]).wait()
                if do_count:
                    @pl.when(qp == 0)
                    def _():
                        pltpu.make_async_copy(ones.at[pl.ds(0, CH)],
                                              cntacc.at[idxd.at[0]],
                                              scs[s]).wait()

            def wait_gather(s):
                pltpu.make_async_copy(tabS.at[idxs.at[0]], rows[s],
                                      sgs[s]).wait()

            def quad(kk, c):
                for s in range(NSLOT):
                    m = kk * NSLOT + s

                    @pl.when(m >= NSLOT)
                    def _(s=s):
                        wait_scatter(s)
                    pltpu.async_copy(tabS.at[idxs.at[m]], rows[s], sgs[s])

                    @pl.when(m >= 1)
                    def _(m=m, s=s):
                        wait_gather((s - 1) % NSLOT)
                        issue_scatter(m - 1, (s - 1) % NSLOT)
                return c
            lax.fori_loop(0, NCHUNK // NSLOT, quad, 0)

            wait_gather(NSLOT - 1)
            issue_scatter(NCHUNK - 1, NSLOT - 1)
            for s in range(NSLOT):
                wait_scatter(s)
            plsc.subcore_barrier()

            rbase = sid * ROWS_PER_SUB
            pltpu.sync_copy(
                acc.at[pl.ds(rbase, ROWS_PER_SUB)],
                part.at[cid, qp, pl.ds(rbase, ROWS_PER_SUB)])
            if do_count:
                @pl.when(qp == 0)
                def _():
                    pltpu.sync_copy(
                        cntacc.at[pl.ds(sid * CSLICE, CSLICE)],
                        cnt.at[cid, sid, 0])
            return carry

        lax.fori_loop(0, 4, do_pass, 0)

    return pl.kernel(
        body, out_type=out_type, mesh=mesh, scratch_types=scratch,
        compiler_params=pltpu.CompilerParams(use_tc_tiling_on_sc=False))


def kernel(x, edge_index_g1, mask_g1, edge_index_g2, mask_g2,
           edge_index_g3_u, mask_g3_u, edge_index_g3_q, mask_g3_q,
           edge_index_g4_u, mask_g4_u, edge_index_g4_q, mask_g4_q,
           Wg1, Wg2, Wg3, Wg4,
           p1_c0_wpl, p1_c0_wpr, p1_c0_bpr, p1_c0_wnl, p1_c0_wnr, p1_c0_bnr,
           p1_c1_wpl, p1_c1_wpr, p1_c1_bpr, p1_c1_wnl, p1_c1_wnr, p1_c1_bnr,
           p2_c0_wpl, p2_c0_wpr, p2_c0_bpr, p2_c0_wnl, p2_c0_wnr, p2_c0_bnr,
           p2_c1_wpl, p2_c1_wpr, p2_c1_bpr, p2_c1_wnl, p2_c1_wnr, p2_c1_bnr):
    f32 = jnp.float32
    eis = [edge_index_g1, edge_index_g2, edge_index_g3_u,
           edge_index_g3_q, edge_index_g4_u, edge_index_g4_q]
    masks = [m.astype(jnp.int32)[None, :] for m in
             (mask_g1, mask_g2, mask_g3_u, mask_g3_q, mask_g4_u, mask_g4_q)]

    w16 = jnp.stack([W[:, q * Q:(q + 1) * Q]
                     for W in (Wg1, Wg2, Wg3, Wg4) for q in range(4)])

    p1c0 = (p1_c0_wpl, p1_c0_wnl, p1_c0_wpr, p1_c0_wnr, p1_c0_bpr, p1_c0_bnr)
    p2c0 = (p2_c0_wpl, p2_c0_wnl, p2_c0_wpr, p2_c0_wnr, p2_c0_bpr, p2_c0_bnr)
    p1c1 = (p1_c1_wpl, p1_c1_wnl, p1_c1_wpr, p1_c1_wnr, p1_c1_bpr, p1_c1_bnr)
    p2c1 = (p2_c1_wpl, p2_c1_wnl, p2_c1_wpr, p2_c1_wnr, p2_c1_bpr, p2_c1_bnr)

    def block_stacks(pA, pB):
        ws = [jnp.stack([pA[i], pB[i]]) for i in range(4)]
        bs = [jnp.stack([pA[i], pB[i]])[:, None, :] for i in (4, 5)]
        return ws + bs

    l1 = [block_stacks(p1c0, p1c0), block_stacks(p2c0, p2c0),
          block_stacks(p2c0, p2c0)]
    l2 = [block_stacks(p1c1, p1c1), block_stacks(p2c1, p2c1),
          block_stacks(p2c1, p2c1)]
    pgls = [(0, 1), (2, 2), (3, 3)]
    tqs = [(0, 4), (8, 8), (12, 12)]

    xq = _tc_proj(x.astype(f32), w16.astype(f32))
    srcs, dks = [], []
    for blk in range(3):
        s_, d_ = _tc_indices(eis[2 * blk:2 * blk + 2],
                             masks[2 * blk:2 * blk + 2])
        srcs.append(s_.reshape(-1, CH))
        dks.append(d_.reshape(-1, CH))

    z2d = jnp.zeros((ROWS_PER_SUB, Q), f32)
    z1d = jnp.zeros((CSLICE,), f32)
    tab1 = xq.reshape(16 * N, Q)

    aggs1 = [_make_agg(tq[0], tq[1], True) for tq in tqs]
    agg2 = _make_agg(0, 4, False)

    def cnt5(c):
        return c.reshape(NB, CNTP)[:, :2 * N].reshape(NB, 2, N, 1)

    parts1, cnts = [], []
    for blk in range(3):
        p_, c_ = aggs1[blk](tab1, srcs[blk], dks[blk], z2d, z1d)
        parts1.append(p_)
        cnts.append(cnt5(c_))
    zsts = [_tc_layer1(parts1[blk], cnts[blk], xq, *l1[blk], pgl=pgls[blk])
            for blk in range(3)]
    parts2 = [agg2(zsts[blk].reshape(NB * 4 * N, Q), srcs[blk],
                   dks[blk], z2d)[0]
              for blk in range(3)]
    outs = [_tc_layer2(parts2[blk], cnts[blk], zsts[blk], *l2[blk])
            for blk in range(3)]

    e1, e2 = outs[0][0], outs[0][1]
    e3 = jnp.concatenate([outs[1][0][:6000], outs[1][1][6000:]], axis=0)
    e4 = jnp.concatenate([outs[2][0][:6000], outs[2][1][6000:]], axis=0)
    return (e1, e2, e3, e4)
